# Initial kernel scaffold; baseline (speedup 1.0000x reference)
import jax
import jax.numpy as jnp
from jax.experimental import pallas as pl


def kernel(potential_energy, potential_energy_uncertainty, z, edge_index, edge_weight, batch_ids, lj_rmins):
    raise NotImplementedError("write your pallas kernel here")


# speedup vs baseline: 234.6462x; 234.6462x over previous
import functools

import jax
import jax.numpy as jnp
from jax import lax
from jax.experimental import pallas as pl
from jax.experimental.pallas import tpu as pltpu
from jax.experimental.pallas import tpu_sc as plsc

N_NODES = 50000
N_EDGES = 1600000
N_GRAPHS = 256
N_ELEM = 100

NUM_CORES = 2
NUM_SUBCORES = 16
NW = NUM_CORES * NUM_SUBCORES
LANES = 16
E_PER_W = N_EDGES // NW
CHUNK = 2000
N_CHUNKS = E_PER_W // CHUNK
VPC = CHUNK // LANES

_mesh = plsc.VectorSubcoreMesh(core_axis_name="c", subcore_axis_name="s")


@functools.partial(
    pl.kernel,
    out_type=jax.ShapeDtypeStruct((NW, N_GRAPHS), jnp.float32),
    mesh=_mesh,
    compiler_params=pltpu.CompilerParams(needs_layout_passes=False),
    scratch_types=[
        pltpu.VMEM((N_NODES,), jnp.int32),
        pltpu.VMEM((N_ELEM, N_ELEM), jnp.float32),
        pltpu.VMEM((CHUNK,), jnp.int32),
        pltpu.VMEM((CHUNK,), jnp.int32),
        pltpu.VMEM((CHUNK,), jnp.float32),
        pltpu.VMEM((LANES * N_GRAPHS,), jnp.float32),
        pltpu.VMEM((N_GRAPHS,), jnp.float32),
    ],
)
def _lj_kernel(src_hbm, dst_hbm, w_hbm, packed_hbm, lj_hbm, out_hbm,
               packed_v, lj_v, src_v, dst_v, w_v, acc_v, part_v):
    wid = lax.axis_index("s") * NUM_CORES + lax.axis_index("c")

    pltpu.sync_copy(packed_hbm, packed_v)
    pltpu.sync_copy(lj_hbm, lj_v)

    zeros16 = jnp.zeros((LANES,), jnp.float32)

    def zero_body(i, carry):
        acc_v[pl.ds(i * LANES, LANES)] = zeros16
        return carry

    lax.fori_loop(0, LANES * N_GRAPHS // LANES, zero_body, 0)

    lane = lax.iota(jnp.int32, LANES)
    ebase = wid * E_PER_W

    def edge_body(i, carry):
        off = i * LANES
        s = src_v[pl.ds(off, LANES)]
        d = dst_v[pl.ds(off, LANES)]
        w = w_v[pl.ds(off, LANES)]
        vs = plsc.load_gather(packed_v, [s])
        vd = plsc.load_gather(packed_v, [d])
        zi = jnp.bitwise_and(vs, 127)
        g = lax.shift_right_logical(vs, 7)
        zj = jnp.bitwise_and(vd, 127)
        rmin = plsc.load_gather(lj_v, [zi, zj])
        x = rmin / w
        x2 = x * x
        x4 = x2 * x2
        x6 = x4 * x2
        x12 = x6 * x6
        val = jnp.where(rmin > w, x12 - 1.0, jnp.zeros_like(x12))
        plsc.addupdate_scatter(acc_v, [lane * N_GRAPHS + g], val)
        return carry

    def chunk_body(c, carry):
        base = pl.multiple_of(ebase + c * CHUNK, 8)
        pltpu.sync_copy(src_hbm.at[pl.ds(base, CHUNK)], src_v)
        pltpu.sync_copy(dst_hbm.at[pl.ds(base, CHUNK)], dst_v)
        pltpu.sync_copy(w_hbm.at[pl.ds(base, CHUNK)], w_v)
        lax.fori_loop(0, VPC, edge_body, carry)
        return carry

    lax.fori_loop(0, N_CHUNKS, chunk_body, 0)

    for j in range(N_GRAPHS // LANES):
        tot = acc_v[pl.ds(0 * N_GRAPHS + j * LANES, LANES)]
        for l in range(1, LANES):
            tot = tot + acc_v[pl.ds(l * N_GRAPHS + j * LANES, LANES)]
        part_v[pl.ds(j * LANES, LANES)] = tot

    pltpu.sync_copy(part_v, out_hbm.at[wid])


def kernel(potential_energy, potential_energy_uncertainty, z, edge_index,
           edge_weight, batch_ids, lj_rmins):
    src = edge_index[0].astype(jnp.int32)
    dst = edge_index[1].astype(jnp.int32)
    packed = (batch_ids.astype(jnp.int32) * 128
              + (z.astype(jnp.int32) - 1))
    partials = _lj_kernel(src, dst, edge_weight, packed,
                          lj_rmins.astype(jnp.float32))
    ljr = jnp.sum(partials, axis=0)[:, None]
    combined = (potential_energy
                - 0.5 * potential_energy_uncertainty
                + ljr)
    return (combined, potential_energy, -potential_energy_uncertainty, ljr)


# speedup vs baseline: 234.8891x; 1.0010x over previous
import functools

import jax
import jax.numpy as jnp
from jax import lax
from jax.experimental import pallas as pl
from jax.experimental.pallas import tpu as pltpu
from jax.experimental.pallas import tpu_sc as plsc

N_NODES = 50000
N_EDGES = 1600000
N_GRAPHS = 256
N_ELEM = 100

NUM_CORES = 2
NUM_SUBCORES = 16
NW = NUM_CORES * NUM_SUBCORES
LANES = 16
E_PER_W = N_EDGES // NW
CHUNK = 2000
N_CHUNKS = E_PER_W // CHUNK
VPC = CHUNK // LANES
UNROLL = 5

_mesh = plsc.VectorSubcoreMesh(core_axis_name="c", subcore_axis_name="s")


@functools.partial(
    pl.kernel,
    out_type=jax.ShapeDtypeStruct((NW, N_GRAPHS), jnp.float32),
    mesh=_mesh,
    compiler_params=pltpu.CompilerParams(needs_layout_passes=False),
    scratch_types=[
        pltpu.VMEM((N_NODES,), jnp.int32),
        pltpu.VMEM((N_ELEM, N_ELEM), jnp.float32),
        pltpu.VMEM((CHUNK,), jnp.int32),
        pltpu.VMEM((CHUNK,), jnp.int32),
        pltpu.VMEM((CHUNK,), jnp.float32),
        pltpu.VMEM((LANES * N_GRAPHS,), jnp.float32),
        pltpu.VMEM((N_GRAPHS,), jnp.float32),
    ],
)
def _lj_kernel(src_hbm, dst_hbm, w_hbm, packed_hbm, lj_hbm, out_hbm,
               packed_v, lj_v, src_v, dst_v, w_v, acc_v, part_v):
    wid = lax.axis_index("s") * NUM_CORES + lax.axis_index("c")

    pltpu.sync_copy(packed_hbm, packed_v)
    pltpu.sync_copy(lj_hbm, lj_v)

    zeros16 = jnp.zeros((LANES,), jnp.float32)

    def zero_body(i, carry):
        acc_v[pl.ds(i * LANES, LANES)] = zeros16
        return carry

    lax.fori_loop(0, LANES * N_GRAPHS // LANES, zero_body, 0)

    lane = lax.iota(jnp.int32, LANES)
    ebase = wid * E_PER_W

    def edge_group_body(i, carry):
        base_off = i * (LANES * UNROLL)
        for u in range(UNROLL):
            off = base_off + u * LANES
            s = src_v[pl.ds(off, LANES)]
            d = dst_v[pl.ds(off, LANES)]
            w = w_v[pl.ds(off, LANES)]
            vs = plsc.load_gather(packed_v, [s])
            vd = plsc.load_gather(packed_v, [d])
            zi = jnp.bitwise_and(vs, 127)
            g = lax.shift_right_logical(vs, 7)
            zj = jnp.bitwise_and(vd, 127)
            rmin = plsc.load_gather(lj_v, [zi, zj])
            x = rmin / w
            x2 = x * x
            x4 = x2 * x2
            x6 = x4 * x2
            x12 = x6 * x6
            val = jnp.where(rmin > w, x12 - 1.0, jnp.zeros_like(x12))
            plsc.addupdate_scatter(acc_v, [lane * N_GRAPHS + g], val)
        return carry

    def chunk_body(c, carry):
        base = pl.multiple_of(ebase + c * CHUNK, 8)
        pltpu.sync_copy(src_hbm.at[pl.ds(base, CHUNK)], src_v)
        pltpu.sync_copy(dst_hbm.at[pl.ds(base, CHUNK)], dst_v)
        pltpu.sync_copy(w_hbm.at[pl.ds(base, CHUNK)], w_v)
        lax.fori_loop(0, VPC // UNROLL, edge_group_body, carry)
        return carry

    lax.fori_loop(0, N_CHUNKS, chunk_body, 0)

    for j in range(N_GRAPHS // LANES):
        tot = acc_v[pl.ds(0 * N_GRAPHS + j * LANES, LANES)]
        for l in range(1, LANES):
            tot = tot + acc_v[pl.ds(l * N_GRAPHS + j * LANES, LANES)]
        part_v[pl.ds(j * LANES, LANES)] = tot

    pltpu.sync_copy(part_v, out_hbm.at[wid])


def kernel(potential_energy, potential_energy_uncertainty, z, edge_index,
           edge_weight, batch_ids, lj_rmins):
    src = edge_index[0].astype(jnp.int32)
    dst = edge_index[1].astype(jnp.int32)
    packed = (batch_ids.astype(jnp.int32) * 128
              + (z.astype(jnp.int32) - 1))
    partials = _lj_kernel(src, dst, edge_weight, packed,
                          lj_rmins.astype(jnp.float32))
    ljr = jnp.sum(partials, axis=0)[:, None]
    combined = (potential_energy
                - 0.5 * potential_energy_uncertainty
                + ljr)
    return (combined, potential_energy, -potential_energy_uncertainty, ljr)


# speedup vs baseline: 295.6536x; 1.2587x over previous
import functools

import jax
import jax.numpy as jnp
from jax import lax
from jax.experimental import pallas as pl
from jax.experimental.pallas import tpu as pltpu
from jax.experimental.pallas import tpu_sc as plsc

N_NODES = 50000
N_EDGES = 1600000
N_GRAPHS = 256
N_ELEM = 100

NUM_CORES = 2
NUM_SUBCORES = 16
NW = NUM_CORES * NUM_SUBCORES
LANES = 16
E_PER_W = N_EDGES // NW
CHUNK = 2000
N_CHUNKS = E_PER_W // CHUNK
VPC = CHUNK // LANES
UNROLL = 5

_mesh = plsc.VectorSubcoreMesh(core_axis_name="c", subcore_axis_name="s")


@functools.partial(
    pl.kernel,
    out_type=jax.ShapeDtypeStruct((NW, N_GRAPHS), jnp.float32),
    mesh=_mesh,
    compiler_params=pltpu.CompilerParams(needs_layout_passes=False),
    scratch_types=[
        pltpu.VMEM((N_NODES,), jnp.int32),
        pltpu.VMEM((N_ELEM, N_ELEM), jnp.float32),
        pltpu.VMEM((CHUNK,), jnp.int32),
        pltpu.VMEM((CHUNK,), jnp.int32),
        pltpu.VMEM((CHUNK,), jnp.float32),
        pltpu.VMEM((CHUNK,), jnp.int32),
        pltpu.VMEM((CHUNK,), jnp.int32),
        pltpu.VMEM((CHUNK,), jnp.float32),
        pltpu.VMEM((LANES * N_GRAPHS,), jnp.float32),
        pltpu.VMEM((N_GRAPHS,), jnp.float32),
        pltpu.SemaphoreType.DMA,
        pltpu.SemaphoreType.DMA,
    ],
)
def _lj_kernel(src_hbm, dst_hbm, w_hbm, packed_hbm, lj_hbm, out_hbm,
               packed_v, lj_v, src_a, dst_a, w_a, src_b, dst_b, w_b,
               acc_v, part_v, sem_a, sem_b):
    wid = lax.axis_index("s") * NUM_CORES + lax.axis_index("c")

    pltpu.sync_copy(packed_hbm, packed_v)
    pltpu.sync_copy(lj_hbm, lj_v)

    zeros16 = jnp.zeros((LANES,), jnp.float32)

    def zero_body(i, carry):
        acc_v[pl.ds(i * LANES, LANES)] = zeros16
        return carry

    lax.fori_loop(0, LANES * N_GRAPHS // LANES, zero_body, 0)

    lane = lax.iota(jnp.int32, LANES)
    ebase = wid * E_PER_W
    slot_a = (src_a, dst_a, w_a, sem_a)
    slot_b = (src_b, dst_b, w_b, sem_b)

    def fire(c, slot):
        sv, dv, wv, sem = slot
        base = pl.multiple_of(ebase + c * CHUNK, 8)
        pltpu.async_copy(src_hbm.at[pl.ds(base, CHUNK)], sv, sem)
        pltpu.async_copy(dst_hbm.at[pl.ds(base, CHUNK)], dv, sem)
        pltpu.async_copy(w_hbm.at[pl.ds(base, CHUNK)], wv, sem)

    def drain(slot):
        sv, dv, wv, sem = slot
        pltpu.make_async_copy(src_hbm.at[pl.ds(0, CHUNK)], sv, sem).wait()
        pltpu.make_async_copy(dst_hbm.at[pl.ds(0, CHUNK)], dv, sem).wait()
        pltpu.make_async_copy(w_hbm.at[pl.ds(0, CHUNK)], wv, sem).wait()

    def compute_chunk(slot, carry):
        sv, dv, wv, _ = slot

        def edge_group_body(i, inner):
            base_off = i * (LANES * UNROLL)
            for u in range(UNROLL):
                off = base_off + u * LANES
                s = sv[pl.ds(off, LANES)]
                d = dv[pl.ds(off, LANES)]
                w = wv[pl.ds(off, LANES)]
                vs = plsc.load_gather(packed_v, [s])
                vd = plsc.load_gather(packed_v, [d])
                zi = jnp.bitwise_and(vs, 127)
                g = lax.shift_right_logical(vs, 7)
                zj = jnp.bitwise_and(vd, 127)
                rmin = plsc.load_gather(lj_v, [zi, zj])
                x = rmin / w
                x2 = x * x
                x4 = x2 * x2
                x6 = x4 * x2
                x12 = x6 * x6
                val = jnp.where(rmin > w, x12 - 1.0, jnp.zeros_like(x12))
                plsc.addupdate_scatter(acc_v, [lane * N_GRAPHS + g], val)
            return inner

        return lax.fori_loop(0, VPC // UNROLL, edge_group_body, carry)

    fire(0, slot_a)

    def pair_body(k, carry):
        c0 = k * 2
        fire(c0 + 1, slot_b)
        drain(slot_a)
        carry = compute_chunk(slot_a, carry)
        fire(c0 + 2, slot_a)
        drain(slot_b)
        carry = compute_chunk(slot_b, carry)
        return carry

    carry = lax.fori_loop(0, (N_CHUNKS - 1) // 2, pair_body, 0)
    drain(slot_a)
    compute_chunk(slot_a, carry)

    for j in range(N_GRAPHS // LANES):
        tot = acc_v[pl.ds(0 * N_GRAPHS + j * LANES, LANES)]
        for l in range(1, LANES):
            tot = tot + acc_v[pl.ds(l * N_GRAPHS + j * LANES, LANES)]
        part_v[pl.ds(j * LANES, LANES)] = tot

    pltpu.sync_copy(part_v, out_hbm.at[wid])


def kernel(potential_energy, potential_energy_uncertainty, z, edge_index,
           edge_weight, batch_ids, lj_rmins):
    src = edge_index[0].astype(jnp.int32)
    dst = edge_index[1].astype(jnp.int32)
    packed = (batch_ids.astype(jnp.int32) * 128
              + (z.astype(jnp.int32) - 1))
    partials = _lj_kernel(src, dst, edge_weight, packed,
                          lj_rmins.astype(jnp.float32))
    ljr = jnp.sum(partials, axis=0)[:, None]
    combined = (potential_energy
                - 0.5 * potential_energy_uncertainty
                + ljr)
    return (combined, potential_energy, -potential_energy_uncertainty, ljr)


# speedup vs baseline: 452.8579x; 1.5317x over previous
import functools

import jax
import jax.numpy as jnp
from jax import lax
from jax.experimental import pallas as pl
from jax.experimental.pallas import tpu as pltpu
from jax.experimental.pallas import tpu_sc as plsc

N_NODES = 50000
N_EDGES = 1600000
N_GRAPHS = 256
N_ELEM = 100

NUM_CORES = 2
NUM_SUBCORES = 16
NW = NUM_CORES * NUM_SUBCORES
LANES = 16
E_PER_W = N_EDGES // NW
CHUNK = 2000
N_CHUNKS = E_PER_W // CHUNK
VPC = CHUNK // LANES
UNROLL = 5

_mesh = plsc.VectorSubcoreMesh(core_axis_name="c", subcore_axis_name="s")


@functools.partial(
    pl.kernel,
    out_type=jax.ShapeDtypeStruct((NW, N_GRAPHS), jnp.float32),
    mesh=_mesh,
    compiler_params=pltpu.CompilerParams(needs_layout_passes=False),
    scratch_types=[
        pltpu.VMEM((N_NODES,), jnp.int32),
        pltpu.VMEM((N_ELEM, N_ELEM), jnp.float32),
        pltpu.VMEM((CHUNK,), jnp.int32),
        pltpu.VMEM((CHUNK,), jnp.int32),
        pltpu.VMEM((CHUNK,), jnp.float32),
        pltpu.VMEM((CHUNK,), jnp.int32),
        pltpu.VMEM((CHUNK,), jnp.int32),
        pltpu.VMEM((CHUNK,), jnp.float32),
        pltpu.VMEM((LANES * N_GRAPHS,), jnp.float32),
        pltpu.VMEM((N_GRAPHS,), jnp.float32),
        pltpu.SemaphoreType.DMA,
        pltpu.SemaphoreType.DMA,
    ],
)
def _lj_kernel(src_hbm, dst_hbm, w_hbm, packed_hbm, lj_hbm, out_hbm,
               packed_v, lj_v, src_a, dst_a, w_a, src_b, dst_b, w_b,
               acc_v, part_v, sem_a, sem_b):
    wid = lax.axis_index("s") * NUM_CORES + lax.axis_index("c")

    pltpu.sync_copy(packed_hbm, packed_v)
    pltpu.sync_copy(lj_hbm, lj_v)

    zeros16 = jnp.zeros((LANES,), jnp.float32)

    def zero_body(i, carry):
        acc_v[pl.ds(i * LANES, LANES)] = zeros16
        return carry

    lax.fori_loop(0, LANES * N_GRAPHS // LANES, zero_body, 0)

    lane = lax.iota(jnp.int32, LANES)
    ebase = wid * E_PER_W
    slot_a = (src_a, dst_a, w_a, sem_a)
    slot_b = (src_b, dst_b, w_b, sem_b)

    def fire(c, slot):
        sv, dv, wv, sem = slot
        base = pl.multiple_of(ebase + c * CHUNK, 8)
        pltpu.async_copy(src_hbm.at[pl.ds(base, CHUNK)], sv, sem)
        pltpu.async_copy(dst_hbm.at[pl.ds(base, CHUNK)], dv, sem)
        pltpu.async_copy(w_hbm.at[pl.ds(base, CHUNK)], wv, sem)

    def drain(slot):
        sv, dv, wv, sem = slot
        pltpu.make_async_copy(src_hbm.at[pl.ds(0, CHUNK)], sv, sem).wait()
        pltpu.make_async_copy(dst_hbm.at[pl.ds(0, CHUNK)], dv, sem).wait()
        pltpu.make_async_copy(w_hbm.at[pl.ds(0, CHUNK)], wv, sem).wait()

    def compute_chunk(slot, carry):
        sv, dv, wv, _ = slot

        @plsc.parallel_loop(0, VPC, unroll=UNROLL)
        def _edge_body(i):
            off = i * LANES
            s = sv[pl.ds(off, LANES)]
            d = dv[pl.ds(off, LANES)]
            w = wv[pl.ds(off, LANES)]
            vs = plsc.load_gather(packed_v, [s])
            vd = plsc.load_gather(packed_v, [d])
            zi = jnp.bitwise_and(vs, 127)
            g = lax.shift_right_logical(vs, 7)
            zj = jnp.bitwise_and(vd, 127)
            rmin = plsc.load_gather(lj_v, [zi, zj])
            x = rmin / w
            x2 = x * x
            x4 = x2 * x2
            x6 = x4 * x2
            x12 = x6 * x6
            val = jnp.where(rmin > w, x12 - 1.0, jnp.zeros_like(x12))
            plsc.addupdate_scatter(acc_v, [lane * N_GRAPHS + g], val)

        return carry

    fire(0, slot_a)

    def pair_body(k, carry):
        c0 = k * 2
        fire(c0 + 1, slot_b)
        drain(slot_a)
        carry = compute_chunk(slot_a, carry)
        fire(c0 + 2, slot_a)
        drain(slot_b)
        carry = compute_chunk(slot_b, carry)
        return carry

    carry = lax.fori_loop(0, (N_CHUNKS - 1) // 2, pair_body, 0)
    drain(slot_a)
    compute_chunk(slot_a, carry)

    for j in range(N_GRAPHS // LANES):
        tot = acc_v[pl.ds(0 * N_GRAPHS + j * LANES, LANES)]
        for l in range(1, LANES):
            tot = tot + acc_v[pl.ds(l * N_GRAPHS + j * LANES, LANES)]
        part_v[pl.ds(j * LANES, LANES)] = tot

    pltpu.sync_copy(part_v, out_hbm.at[wid])


def kernel(potential_energy, potential_energy_uncertainty, z, edge_index,
           edge_weight, batch_ids, lj_rmins):
    src = edge_index[0].astype(jnp.int32)
    dst = edge_index[1].astype(jnp.int32)
    packed = (batch_ids.astype(jnp.int32) * 128
              + (z.astype(jnp.int32) - 1))
    partials = _lj_kernel(src, dst, edge_weight, packed,
                          lj_rmins.astype(jnp.float32))
    ljr = jnp.sum(partials, axis=0)[:, None]
    combined = (potential_energy
                - 0.5 * potential_energy_uncertainty
                + ljr)
    return (combined, potential_energy, -potential_energy_uncertainty, ljr)


# speedup vs baseline: 455.0454x; 1.0048x over previous
import functools

import jax
import jax.numpy as jnp
from jax import lax
from jax.experimental import pallas as pl
from jax.experimental.pallas import tpu as pltpu
from jax.experimental.pallas import tpu_sc as plsc

N_NODES = 50000
N_EDGES = 1600000
N_GRAPHS = 256
N_ELEM = 100

NUM_CORES = 2
NUM_SUBCORES = 16
NW = NUM_CORES * NUM_SUBCORES
LANES = 16
E_PER_W = N_EDGES // NW
CHUNK = 10000
N_CHUNKS = E_PER_W // CHUNK
VPC = CHUNK // LANES
UNROLL = 5

_mesh = plsc.VectorSubcoreMesh(core_axis_name="c", subcore_axis_name="s")


@functools.partial(
    pl.kernel,
    out_type=jax.ShapeDtypeStruct((NW, N_GRAPHS), jnp.float32),
    mesh=_mesh,
    compiler_params=pltpu.CompilerParams(needs_layout_passes=False),
    scratch_types=[
        pltpu.VMEM((N_NODES,), jnp.int32),
        pltpu.VMEM((N_ELEM, N_ELEM), jnp.float32),
        pltpu.VMEM((CHUNK,), jnp.int32),
        pltpu.VMEM((CHUNK,), jnp.int32),
        pltpu.VMEM((CHUNK,), jnp.float32),
        pltpu.VMEM((CHUNK,), jnp.int32),
        pltpu.VMEM((CHUNK,), jnp.int32),
        pltpu.VMEM((CHUNK,), jnp.float32),
        pltpu.VMEM((LANES * N_GRAPHS,), jnp.float32),
        pltpu.VMEM((N_GRAPHS,), jnp.float32),
        pltpu.SemaphoreType.DMA,
        pltpu.SemaphoreType.DMA,
    ],
)
def _lj_kernel(src_hbm, dst_hbm, w_hbm, packed_hbm, lj_hbm, out_hbm,
               packed_v, lj_v, src_a, dst_a, w_a, src_b, dst_b, w_b,
               acc_v, part_v, sem_a, sem_b):
    wid = lax.axis_index("s") * NUM_CORES + lax.axis_index("c")

    pltpu.sync_copy(packed_hbm, packed_v)
    pltpu.sync_copy(lj_hbm, lj_v)

    zeros16 = jnp.zeros((LANES,), jnp.float32)

    def zero_body(i, carry):
        acc_v[pl.ds(i * LANES, LANES)] = zeros16
        return carry

    lax.fori_loop(0, LANES * N_GRAPHS // LANES, zero_body, 0)

    lane = lax.iota(jnp.int32, LANES)
    ebase = wid * E_PER_W
    slot_a = (src_a, dst_a, w_a, sem_a)
    slot_b = (src_b, dst_b, w_b, sem_b)

    def fire(c, slot):
        sv, dv, wv, sem = slot
        base = pl.multiple_of(ebase + c * CHUNK, 8)
        pltpu.async_copy(src_hbm.at[pl.ds(base, CHUNK)], sv, sem)
        pltpu.async_copy(dst_hbm.at[pl.ds(base, CHUNK)], dv, sem)
        pltpu.async_copy(w_hbm.at[pl.ds(base, CHUNK)], wv, sem)

    def drain(slot):
        sv, dv, wv, sem = slot
        pltpu.make_async_copy(src_hbm.at[pl.ds(0, CHUNK)], sv, sem).wait()
        pltpu.make_async_copy(dst_hbm.at[pl.ds(0, CHUNK)], dv, sem).wait()
        pltpu.make_async_copy(w_hbm.at[pl.ds(0, CHUNK)], wv, sem).wait()

    def compute_chunk(slot, carry):
        sv, dv, wv, _ = slot

        @plsc.parallel_loop(0, VPC, unroll=UNROLL)
        def _edge_body(i):
            off = i * LANES
            s = sv[pl.ds(off, LANES)]
            d = dv[pl.ds(off, LANES)]
            w = wv[pl.ds(off, LANES)]
            vs = plsc.load_gather(packed_v, [s])
            vd = plsc.load_gather(packed_v, [d])
            zi = jnp.bitwise_and(vs, 127)
            g = lax.shift_right_logical(vs, 7)
            zj = jnp.bitwise_and(vd, 127)
            rmin = plsc.load_gather(lj_v, [zi, zj])
            x = rmin / w
            x2 = x * x
            x4 = x2 * x2
            x6 = x4 * x2
            x12 = x6 * x6
            val = jnp.where(rmin > w, x12 - 1.0, jnp.zeros_like(x12))
            plsc.addupdate_scatter(acc_v, [lane * N_GRAPHS + g], val)

        return carry

    fire(0, slot_a)

    def pair_body(k, carry):
        c0 = k * 2
        fire(c0 + 1, slot_b)
        drain(slot_a)
        carry = compute_chunk(slot_a, carry)
        fire(c0 + 2, slot_a)
        drain(slot_b)
        carry = compute_chunk(slot_b, carry)
        return carry

    carry = lax.fori_loop(0, (N_CHUNKS - 1) // 2, pair_body, 0)
    drain(slot_a)
    compute_chunk(slot_a, carry)

    for j in range(N_GRAPHS // LANES):
        tot = acc_v[pl.ds(0 * N_GRAPHS + j * LANES, LANES)]
        for l in range(1, LANES):
            tot = tot + acc_v[pl.ds(l * N_GRAPHS + j * LANES, LANES)]
        part_v[pl.ds(j * LANES, LANES)] = tot

    pltpu.sync_copy(part_v, out_hbm.at[wid])


def kernel(potential_energy, potential_energy_uncertainty, z, edge_index,
           edge_weight, batch_ids, lj_rmins):
    src = edge_index[0].astype(jnp.int32)
    dst = edge_index[1].astype(jnp.int32)
    packed = (batch_ids.astype(jnp.int32) * 128
              + (z.astype(jnp.int32) - 1))
    partials = _lj_kernel(src, dst, edge_weight, packed,
                          lj_rmins.astype(jnp.float32))
    ljr = jnp.sum(partials, axis=0)[:, None]
    combined = (potential_energy
                - 0.5 * potential_energy_uncertainty
                + ljr)
    return (combined, potential_energy, -potential_energy_uncertainty, ljr)
